# Initial kernel scaffold; baseline (speedup 1.0000x reference)
import jax
import jax.numpy as jnp
from jax.experimental import pallas as pl


def kernel(x, norm, params, edge_index):
    raise NotImplementedError("write your pallas kernel here")


# speedup vs baseline: 4.5089x; 4.5089x over previous
import functools

import jax
import jax.numpy as jnp
from jax import lax
from jax.experimental import pallas as pl
from jax.experimental.pallas import tpu as pltpu
from jax.experimental.pallas import tpu_sc as plsc

NSEG = 10000
NEDGE = 320000
D = 128
NCORES = 2
NSUB = 16
NW = NCORES * NSUB
K = 128
NCHUNK = NEDGE // K
RSUB = 632
NSEGA = NSUB * RSUB
NSEGP = 16384
HRS = 8

def _segsum_body(tab_hbm, gidx_hbm, sidx_hbm, nrm_hbm, out_hbm,
                 acc_sh, ig_v, is_v, nr_v, rows_v, zb_v, gsem):
    c = lax.axis_index("c")
    s = lax.axis_index("s")
    wid = s * NCORES + c
    zero16 = jnp.zeros((16,), jnp.float32)

    for r in range(8):
        for j in range(8):
            zb_v[r, pl.ds(j * 16, 16)] = zero16

    def zcopy(j, _):
        off = pl.multiple_of(s * RSUB + j * 8, 8)
        pltpu.sync_copy(zb_v, acc_sh.at[pl.ds(off, 8)])
        return 0
    lax.fori_loop(0, RSUB // 8, zcopy, 0)
    plsc.subcore_barrier()

    niter = (NCHUNK - wid + NW - 1) // NW

    def chunk(i, _):
        g = wid + i * NW
        base = pl.multiple_of(g * K, K)
        pltpu.sync_copy(gidx_hbm.at[pl.ds(base, K)], ig_v)
        pltpu.sync_copy(sidx_hbm.at[pl.ds(base, K)], is_v)
        pltpu.sync_copy(nrm_hbm.at[pl.ds(base, K)], nr_v)
        pltpu.async_copy(tab_hbm.at[ig_v], rows_v, gsem).wait()

        def scale(t, _):
            nv = nr_v[pl.ds(t * 16, 16)]
            for l in range(16):
                nk = nv[l]
                k = t * 16 + l
                for j in range(8):
                    sl = (k, pl.ds(j * 16, 16))
                    rows_v[sl] = rows_v[sl] * nk
            return 0
        lax.fori_loop(0, K // 16, scale, 0)
        pltpu.sync_copy(rows_v, acc_sh.at[is_v], add=True)
        return 0
    lax.fori_loop(0, niter, chunk, 0)

    plsc.subcore_barrier()
    pltpu.sync_copy(acc_sh.at[pl.ds(s * RSUB, RSUB)],
                    out_hbm.at[c, pl.ds(s * RSUB, RSUB)])


@functools.cache
def _get_sc_segsum():
    return pl.kernel(
        _segsum_body,
        out_type=jax.ShapeDtypeStruct((NCORES, NSEGA, D), jnp.float32),
        mesh=plsc.VectorSubcoreMesh(core_axis_name="c", subcore_axis_name="s"),
        compiler_params=pltpu.CompilerParams(needs_layout_passes=False),
        scratch_types=[
            pltpu.VMEM_SHARED((NSEGA, D), jnp.float32),
            pltpu.VMEM((K,), jnp.int32),
            pltpu.VMEM((K,), jnp.int32),
            pltpu.VMEM((K,), jnp.float32),
            pltpu.VMEM((K, D), jnp.float32),
            pltpu.VMEM((8, D), jnp.float32),
            pltpu.SemaphoreType.DMA,
        ],
    )


def _sc_segsum(tab, gidx, sidx, nrm):
    return _get_sc_segsum()(tab, gidx, sidx, nrm)


def _counts_body(d_hbm, s_hbm, out_hbm, h0_v, h1_v, call_sh, tb_v, ob_v,
                 b0_v, b1_v):
    c = lax.axis_index("c")
    s = lax.axis_index("s")
    wid = s * NCORES + c
    zero16 = jnp.zeros((16,), jnp.float32)
    ones16 = jnp.ones((16,), jnp.float32)

    def zrow(r, _):
        for j in range(8):
            h0_v[r, pl.ds(j * 16, 16)] = zero16
            h1_v[r, pl.ds(j * 16, 16)] = zero16
        return 0
    lax.fori_loop(0, 128, zrow, 0)

    niter = (NCHUNK - wid + NW - 1) // NW

    def chunk(i, _):
        g = wid + i * NW
        base = pl.multiple_of(g * K, K)
        pltpu.sync_copy(d_hbm.at[pl.ds(base, K)], b0_v)
        pltpu.sync_copy(s_hbm.at[pl.ds(base, K)], b1_v)
        for j in range(8):
            dv = b0_v[pl.ds(j * 16, 16)]
            plsc.addupdate_scatter(
                h0_v, [lax.shift_right_logical(dv, 7),
                       lax.bitwise_and(dv, 127)], ones16)
            sv = b1_v[pl.ds(j * 16, 16)]
            plsc.addupdate_scatter(
                h1_v, [lax.shift_right_logical(sv, 7),
                       lax.bitwise_and(sv, 127)], ones16)
        return 0
    lax.fori_loop(0, niter, chunk, 0)

    pltpu.sync_copy(h0_v, call_sh.at[s * 2])
    pltpu.sync_copy(h1_v, call_sh.at[s * 2 + 1])
    plsc.subcore_barrier()

    for d in range(2):
        for t in range(NSUB):
            pltpu.sync_copy(call_sh.at[t * 2 + d, pl.ds(s * HRS, HRS)],
                            tb_v.at[t])

        def red(i, _):
            r = i // 8
            j = i % 8
            v = tb_v[0, r, pl.ds(j * 16, 16)]
            for t in range(1, NSUB):
                v = v + tb_v[t, r, pl.ds(j * 16, 16)]
            ob_v[r, pl.ds(j * 16, 16)] = v
            return 0
        lax.fori_loop(0, HRS * 8, red, 0)
        pltpu.sync_copy(ob_v, out_hbm.at[c, d, s])


@functools.cache
def _get_sc_counts():
    return pl.kernel(
        _counts_body,
        out_type=jax.ShapeDtypeStruct((NCORES, 2, NSUB, HRS, 128),
                                      jnp.float32),
        mesh=plsc.VectorSubcoreMesh(core_axis_name="c", subcore_axis_name="s"),
        compiler_params=pltpu.CompilerParams(needs_layout_passes=False),
        scratch_types=[
            pltpu.VMEM((128, 128), jnp.float32),
            pltpu.VMEM((128, 128), jnp.float32),
            pltpu.VMEM_SHARED((NSUB * 2, 128, 128), jnp.float32),
            pltpu.VMEM((NSUB, HRS, 128), jnp.float32),
            pltpu.VMEM((HRS, 128), jnp.float32),
            pltpu.VMEM((K,), jnp.int32),
            pltpu.VMEM((K,), jnp.int32),
        ],
    )


def _sc_counts(dst, src):
    return _get_sc_counts()(dst, src)


_R = 1000


def _mlp_body(x_ref, w1_ref, b1_ref, w2_ref, b2_ref, o_ref, *, final_relu):
    h = jnp.dot(x_ref[...], w1_ref[...], preferred_element_type=jnp.float32)
    h = jnp.maximum(h + b1_ref[...], 0.0)
    o = jnp.dot(h, w2_ref[...], preferred_element_type=jnp.float32)
    o = o + b2_ref[...]
    if final_relu:
        o = jnp.maximum(o, 0.0)
    o_ref[...] = o


def _mlp_tc(x, p, final_relu):
    (w1, b1), (w2, b2) = p
    n, d = x.shape
    dh = w1.shape[1]
    do = w2.shape[1]
    return pl.pallas_call(
        functools.partial(_mlp_body, final_relu=final_relu),
        grid=(n // _R,),
        in_specs=[
            pl.BlockSpec((_R, d), lambda i: (i, 0)),
            pl.BlockSpec((d, dh), lambda i: (0, 0)),
            pl.BlockSpec((1, dh), lambda i: (0, 0)),
            pl.BlockSpec((dh, do), lambda i: (0, 0)),
            pl.BlockSpec((1, do), lambda i: (0, 0)),
        ],
        out_specs=pl.BlockSpec((_R, do), lambda i: (i, 0)),
        out_shape=jax.ShapeDtypeStruct((n, do), jnp.float32),
    )(x, w1, b1.reshape(1, -1), w2, b2.reshape(1, -1))


def _dec_body(p0_ref, p1_ref, c0_ref, c1_ref, w1_ref, b1_ref, w2_ref, b2_ref,
              o_ref):
    cnt = c0_ref[...] + c1_ref[...]
    inv = 1.0 / jnp.maximum(cnt, 1.0)
    xin = (p0_ref[...] + p1_ref[...]) * inv
    h = jnp.dot(xin, w1_ref[...], preferred_element_type=jnp.float32)
    h = jnp.maximum(h + b1_ref[...], 0.0)
    o = jnp.dot(h, w2_ref[...], preferred_element_type=jnp.float32)
    o_ref[...] = jnp.maximum(o + b2_ref[...], 0.0)


def _dec_tc(p0, p1, c0, c1, p):
    (w1, b1), (w2, b2) = p
    n, d = p0.shape
    dh = w1.shape[1]
    do = w2.shape[1]
    return pl.pallas_call(
        _dec_body,
        grid=(n // _R,),
        in_specs=[
            pl.BlockSpec((_R, d), lambda i: (i, 0)),
            pl.BlockSpec((_R, d), lambda i: (i, 0)),
            pl.BlockSpec((_R, 1), lambda i: (i, 0)),
            pl.BlockSpec((_R, 1), lambda i: (i, 0)),
            pl.BlockSpec((d, dh), lambda i: (0, 0)),
            pl.BlockSpec((1, dh), lambda i: (0, 0)),
            pl.BlockSpec((dh, do), lambda i: (0, 0)),
            pl.BlockSpec((1, do), lambda i: (0, 0)),
        ],
        out_specs=pl.BlockSpec((_R, do), lambda i: (i, 0)),
        out_shape=jax.ShapeDtypeStruct((n, do), jnp.float32),
    )(p0, p1, c0, c1, w1, b1.reshape(1, -1), w2, b2.reshape(1, -1))


def kernel(x, norm, params, edge_index):
    src = edge_index[0].astype(jnp.int32)
    cidx = jnp.min(edge_index[1])
    dst = (edge_index[1] - cidx).astype(jnp.int32)

    counts = _sc_counts(dst, src).reshape(NCORES, 2, NSEGP)
    cd0 = counts[0, 0, :NSEG].reshape(NSEG, 1)
    cd1 = counts[1, 0, :NSEG].reshape(NSEG, 1)
    cs0 = counts[0, 1, :NSEG].reshape(NSEG, 1)
    cs1 = counts[1, 1, :NSEG].reshape(NSEG, 1)

    h = x
    for i in range(2):
        pv = params['v2e'][i]
        pe = params['e2v'][i]
        t = _mlp_tc(h, pv['enc'], final_relu=True)
        parts = _sc_segsum(t, src, dst, norm)
        e = _dec_tc(parts[0, :NSEG], parts[1, :NSEG], cd0, cd1, pv['dec'])
        t2 = _mlp_tc(e, pe['enc'], final_relu=True)
        parts2 = _sc_segsum(t2, dst, src, norm)
        h = _dec_tc(parts2[0, :NSEG], parts2[1, :NSEG], cs0, cs1, pe['dec'])
    return _mlp_tc(h, params['clf'], final_relu=False)
